# baseline (speedup 1.0000x reference)
import functools

import jax
import jax.numpy as jnp
from jax.experimental import pallas as pl
from jax.experimental.pallas import tpu as pltpu

_VMEM_LIMIT = 100 * 1024 * 1024


def _mlp_kernel(x_ref, w0_ref, b0_ref, w1_ref, b1_ref, w2_ref, b2_ref, o_ref):
    t = x_ref[...].astype(jnp.bfloat16)
    t = jnp.dot(t, w0_ref[...], preferred_element_type=jnp.float32) + b0_ref[...]
    t = jnp.maximum(t, 0.0).astype(jnp.bfloat16)
    t = jnp.dot(t, w1_ref[...], preferred_element_type=jnp.float32) + b1_ref[...]
    t = jnp.maximum(t, 0.0).astype(jnp.bfloat16)
    t = jnp.dot(t, w2_ref[...], preferred_element_type=jnp.float32) + b2_ref[...]
    o_ref[...] = jnp.maximum(t, 0.0)


def _mlp(x, w0, b0, w1, b1, w2, b2, *, tm):
    n, fin = x.shape
    f0, f1, f2 = w0.shape[1], w1.shape[1], w2.shape[1]
    tm = min(tm, n)
    grid = (n // tm,)
    return pl.pallas_call(
        _mlp_kernel,
        out_shape=jax.ShapeDtypeStruct((n, f2), jnp.float32),
        grid=grid,
        in_specs=[
            pl.BlockSpec((tm, fin), lambda i: (i, 0)),
            pl.BlockSpec((fin, f0), lambda i: (0, 0)),
            pl.BlockSpec((1, f0), lambda i: (0, 0)),
            pl.BlockSpec((f0, f1), lambda i: (0, 0)),
            pl.BlockSpec((1, f1), lambda i: (0, 0)),
            pl.BlockSpec((f1, f2), lambda i: (0, 0)),
            pl.BlockSpec((1, f2), lambda i: (0, 0)),
        ],
        out_specs=pl.BlockSpec((tm, f2), lambda i: (i, 0)),
        compiler_params=pltpu.CompilerParams(
            dimension_semantics=("parallel",),
            vmem_limit_bytes=_VMEM_LIMIT,
        ),
    )(x, w0, b0, w1, b1, w2, b2)


def _prop2_kernel(c_ref, x0_ref, o_ref, cbf_ref, dinv_ref, h1_ref,
                  *, alpha, tm, nsub):
    i = pl.program_id(0)
    nblk = pl.num_programs(0)
    deg = jnp.sum(c_ref[0:tm, :], axis=1, keepdims=True)
    for k in range(1, nsub):
        deg += jnp.sum(c_ref[k * tm:(k + 1) * tm, :], axis=1, keepdims=True)
    dinv_ref[pl.ds(i * tm, tm), :] = jax.lax.rsqrt(deg + 1.0)
    cbf_ref[pl.ds(i * tm * nsub, tm * nsub), :] = c_ref[...].astype(jnp.bfloat16)

    @pl.when(i == nblk - 1)
    def _steps():
        def one_step(src_ref, out_ref):
            g = (src_ref[...] * dinv_ref[...]).astype(jnp.bfloat16)

            def body(j, _):
                a = jnp.dot(cbf_ref[pl.ds(j * tm * nsub, tm), :], g[0:128, :],
                            preferred_element_type=jnp.float32)
                for k in range(1, nsub):
                    a += jnp.dot(
                        cbf_ref[pl.ds(j * tm * nsub + k * tm, tm), :],
                        g[k * 128:(k + 1) * 128, :],
                        preferred_element_type=jnp.float32)
                dinv_j = dinv_ref[pl.ds(j * tm, tm), :]
                g_j = src_ref[pl.ds(j * tm, tm), :] * dinv_j
                out_ref[pl.ds(j * tm, tm), :] = (
                    (1.0 - alpha) * dinv_j * (a + g_j)
                    + alpha * x0_ref[pl.ds(j * tm, tm), :])
                return 0

            jax.lax.fori_loop(0, nblk, body, 0)

        one_step(x0_ref, h1_ref)
        one_step(h1_ref, o_ref)


def _propagate(counts, x0, *, alpha, tm):
    n, f = x0.shape
    tm = min(tm, n)
    nsub = n // 128
    grid = (n // tm,)
    return pl.pallas_call(
        functools.partial(_prop2_kernel, alpha=alpha, tm=tm, nsub=nsub),
        out_shape=jax.ShapeDtypeStruct((n, f), jnp.float32),
        grid=grid,
        in_specs=[
            pl.BlockSpec((tm * nsub, 128), lambda i: (i, 0)),
            pl.BlockSpec((n, f), lambda i: (0, 0)),
        ],
        out_specs=pl.BlockSpec((n, f), lambda i: (0, 0)),
        scratch_shapes=[
            pltpu.VMEM((n * nsub, 128), jnp.bfloat16),
            pltpu.VMEM((n, 1), jnp.float32),
            pltpu.VMEM((n, f), jnp.float32),
        ],
        compiler_params=pltpu.CompilerParams(
            dimension_semantics=("arbitrary",),
            vmem_limit_bytes=_VMEM_LIMIT,
        ),
    )(counts, x0)


def kernel(x, edge_index, w0, w1, w2, b0, b1, b2):
    n = x.shape[0]
    alpha = 0.1
    k_steps = 2

    x0 = _mlp(
        x,
        w0.astype(jnp.bfloat16), b0,
        w1.astype(jnp.bfloat16), b1,
        w2.astype(jnp.bfloat16), b2,
        tm=1024,
    )

    src = edge_index[0]
    dst = edge_index[1]
    tm = min(512, n)
    nsub = n // 128
    row = (dst // tm) * (tm * nsub) + (src // 128) * tm + (dst % tm)
    pos = row * 128 + (src % 128)
    pos_sorted = jnp.sort(pos)
    ones = jnp.ones((dst.shape[0],), jnp.float32)
    flat = jnp.zeros((n * n,), jnp.float32).at[pos_sorted].add(
        ones, indices_are_sorted=True)
    counts_slabs = flat.reshape(n * nsub, 128)

    del k_steps
    return _propagate(counts_slabs, x0, alpha=alpha, tm=tm)


# speedup vs baseline: 1.0328x; 1.0328x over previous
import functools

import jax
import jax.numpy as jnp
from jax.experimental import pallas as pl
from jax.experimental.pallas import tpu as pltpu

_VMEM_LIMIT = 100 * 1024 * 1024


def _mlp_kernel(x_ref, w0_ref, b0_ref, w1_ref, b1_ref, w2_ref, b2_ref, o_ref):
    t = x_ref[...].astype(jnp.bfloat16)
    t = jnp.dot(t, w0_ref[...], preferred_element_type=jnp.float32) + b0_ref[...]
    t = jnp.maximum(t, 0.0).astype(jnp.bfloat16)
    t = jnp.dot(t, w1_ref[...], preferred_element_type=jnp.float32) + b1_ref[...]
    t = jnp.maximum(t, 0.0).astype(jnp.bfloat16)
    t = jnp.dot(t, w2_ref[...], preferred_element_type=jnp.float32) + b2_ref[...]
    o_ref[...] = jnp.maximum(t, 0.0)


def _mlp(x, w0, b0, w1, b1, w2, b2, *, tm):
    n, fin = x.shape
    f0, f1, f2 = w0.shape[1], w1.shape[1], w2.shape[1]
    tm = min(tm, n)
    grid = (n // tm,)
    return pl.pallas_call(
        _mlp_kernel,
        out_shape=jax.ShapeDtypeStruct((n, f2), jnp.float32),
        grid=grid,
        in_specs=[
            pl.BlockSpec((tm, fin), lambda i: (i, 0)),
            pl.BlockSpec((fin, f0), lambda i: (0, 0)),
            pl.BlockSpec((1, f0), lambda i: (0, 0)),
            pl.BlockSpec((f0, f1), lambda i: (0, 0)),
            pl.BlockSpec((1, f1), lambda i: (0, 0)),
            pl.BlockSpec((f1, f2), lambda i: (0, 0)),
            pl.BlockSpec((1, f2), lambda i: (0, 0)),
        ],
        out_specs=pl.BlockSpec((tm, f2), lambda i: (i, 0)),
        compiler_params=pltpu.CompilerParams(
            dimension_semantics=("parallel",),
            vmem_limit_bytes=_VMEM_LIMIT,
        ),
    )(x, w0, b0, w1, b1, w2, b2)


def _prop2_kernel(c_ref, x0_ref, o_ref, cbf_ref, dinv_ref, h1_ref,
                  *, alpha, tm, nsub):
    i = pl.program_id(0)
    nblk = pl.num_programs(0)
    deg = jnp.sum(c_ref[0:tm, :], axis=1, keepdims=True)
    for k in range(1, nsub):
        deg += jnp.sum(c_ref[k * tm:(k + 1) * tm, :], axis=1, keepdims=True)
    dinv_ref[pl.ds(i * tm, tm), :] = jax.lax.rsqrt(deg + 1.0)
    cbf_ref[pl.ds(i * tm * nsub, tm * nsub), :] = c_ref[...].astype(jnp.bfloat16)

    @pl.when(i == nblk - 1)
    def _steps():
        def one_step(src_ref, out_ref):
            g = (src_ref[...] * dinv_ref[...]).astype(jnp.bfloat16)

            def body(j, _):
                a = jnp.dot(cbf_ref[pl.ds(j * tm * nsub, tm), :], g[0:128, :],
                            preferred_element_type=jnp.float32)
                for k in range(1, nsub):
                    a += jnp.dot(
                        cbf_ref[pl.ds(j * tm * nsub + k * tm, tm), :],
                        g[k * 128:(k + 1) * 128, :],
                        preferred_element_type=jnp.float32)
                dinv_j = dinv_ref[pl.ds(j * tm, tm), :]
                g_j = src_ref[pl.ds(j * tm, tm), :] * dinv_j
                out_ref[pl.ds(j * tm, tm), :] = (
                    (1.0 - alpha) * dinv_j * (a + g_j)
                    + alpha * x0_ref[pl.ds(j * tm, tm), :])
                return 0

            jax.lax.fori_loop(0, nblk, body, 0)

        one_step(x0_ref, h1_ref)
        one_step(h1_ref, o_ref)


def _propagate(counts, x0, *, alpha, tm):
    n, f = x0.shape
    tm = min(tm, n)
    nsub = n // 128
    grid = (n // tm,)
    return pl.pallas_call(
        functools.partial(_prop2_kernel, alpha=alpha, tm=tm, nsub=nsub),
        out_shape=jax.ShapeDtypeStruct((n, f), jnp.float32),
        grid=grid,
        in_specs=[
            pl.BlockSpec((tm * nsub, 128), lambda i: (i, 0)),
            pl.BlockSpec((n, f), lambda i: (0, 0)),
        ],
        out_specs=pl.BlockSpec((n, f), lambda i: (0, 0)),
        scratch_shapes=[
            pltpu.VMEM((n * nsub, 128), jnp.bfloat16),
            pltpu.VMEM((n, 1), jnp.float32),
            pltpu.VMEM((n, f), jnp.float32),
        ],
        compiler_params=pltpu.CompilerParams(
            dimension_semantics=("arbitrary",),
            vmem_limit_bytes=_VMEM_LIMIT,
        ),
    )(counts, x0)


def kernel(x, edge_index, w0, w1, w2, b0, b1, b2):
    n = x.shape[0]
    alpha = 0.1
    k_steps = 2

    x0 = _mlp(
        x,
        w0.astype(jnp.bfloat16), b0,
        w1.astype(jnp.bfloat16), b1,
        w2.astype(jnp.bfloat16), b2,
        tm=1024,
    )

    src = edge_index[0]
    dst = edge_index[1]
    tm = min(512, n)
    nsub = n // 128
    row = (dst // tm) * (tm * nsub) + (src // 128) * tm + (dst % tm)
    pos = row * 128 + (src % 128)
    ones = jnp.ones((dst.shape[0],), jnp.float32)
    flat = jnp.zeros((n * n,), jnp.float32).at[pos].add(ones)
    counts_slabs = flat.reshape(n * nsub, 128)

    del k_steps
    return _propagate(counts_slabs, x0, alpha=alpha, tm=tm)
